# Initial kernel scaffold; baseline (speedup 1.0000x reference)
import functools

import jax
import jax.numpy as jnp
from jax import lax
from jax.experimental import pallas as pl
from jax.experimental.pallas import tpu as pltpu
from jax.experimental.pallas import tpu_sc as plsc

N = 10000
E = 320000
D_IN = 128
D_HID = 128
D_OUT = 64

NC = 2
NS = 16
L = 16
NW = NC * NS
EPT = E // NW
B = 80
RPS = N // NS


def _make_spmm(D):
    mesh = plsc.VectorSubcoreMesh(core_axis_name="c", subcore_axis_name="s")

    @functools.partial(
        pl.kernel,
        out_type=jax.ShapeDtypeStruct((NC, N, D), jnp.float32),
        mesh=mesh,
        scratch_types=[
            pltpu.VMEM((B,), jnp.int32),
            pltpu.VMEM((B,), jnp.int32),
            pltpu.VMEM((B,), jnp.float32),
            pltpu.VMEM((B, D), jnp.float32),
            pltpu.VMEM_SHARED((N, D), jnp.float32),
        ],
    )
    def spmm(h_hbm, src_hbm, dst_hbm, ev_hbm, z_hbm, out_hbm,
             src_v, dst_v, ev_v, rows_v, acc):
        cid = lax.axis_index("c")
        sid = lax.axis_index("s")

        pltpu.sync_copy(z_hbm, acc.at[pl.ds(sid * RPS, RPS)])
        plsc.subcore_barrier()

        base0 = (cid * NS + sid) * EPT

        @pl.loop(0, EPT, step=B)
        def _(off):
            base = base0 + off
            pltpu.sync_copy(src_hbm.at[pl.ds(base, B)], src_v)
            pltpu.sync_copy(dst_hbm.at[pl.ds(base, B)], dst_v)
            pltpu.sync_copy(ev_hbm.at[pl.ds(base, B)], ev_v)
            pltpu.sync_copy(h_hbm.at[src_v], rows_v)

            @pl.loop(0, B)
            def _(i):
                evb = plsc.load_gather(ev_v, [lax.broadcast(i, (L,))])
                for j in range(D // L):
                    sl = pl.ds(j * L, L)
                    rows_v[i, sl] = rows_v[i, sl] * evb

            pltpu.sync_copy(rows_v, acc.at[dst_v], add=True)

        plsc.subcore_barrier()
        pltpu.sync_copy(acc.at[pl.ds(sid * RPS, RPS)],
                        out_hbm.at[cid].at[pl.ds(sid * RPS, RPS)])

    return spmm


_spmm_128 = _make_spmm(D_HID)
_spmm_64 = _make_spmm(D_OUT)


def _mm1_body(x_ref, w_ref, o_ref):
    o_ref[...] = jnp.dot(x_ref[...], w_ref[...],
                         preferred_element_type=jnp.float32)


def _mid_body(p_ref, b1_ref, w2_ref, o_ref):
    h1 = jnp.maximum(p_ref[0] + p_ref[1] + b1_ref[...], 0.0)
    o_ref[...] = jnp.dot(h1, w2_ref[...], preferred_element_type=jnp.float32)


def _final_body(q_ref, b2_ref, o_ref):
    o_ref[...] = q_ref[0] + q_ref[1] + b2_ref[...]


def kernel(x, edge_index, edge_vals, w1, b1, w2, b2):
    src = edge_index[0]
    dst = edge_index[1]
    z128 = jnp.zeros((RPS, D_HID), jnp.float32)
    z64 = jnp.zeros((RPS, D_OUT), jnp.float32)

    h = pl.pallas_call(
        _mm1_body,
        out_shape=jax.ShapeDtypeStruct((N, D_HID), jnp.float32),
    )(x, w1)

    p = _spmm_128(h, src, dst, edge_vals, z128)

    h2 = pl.pallas_call(
        _mid_body,
        out_shape=jax.ShapeDtypeStruct((N, D_OUT), jnp.float32),
    )(p, b1, w2)

    q = _spmm_64(h2, src, dst, edge_vals, z64)

    out = pl.pallas_call(
        _final_body,
        out_shape=jax.ShapeDtypeStruct((N, D_OUT), jnp.float32),
    )(q, b2)
    return out


# speedup vs baseline: 9.4571x; 9.4571x over previous
import dataclasses
import functools

import jax
import jax.numpy as jnp
from jax import lax
from jax.experimental import pallas as pl
from jax.experimental.pallas import tpu as pltpu
from jax.experimental.pallas import tpu_sc as plsc

N = 10000
E = 320000
D_IN = 128
D_HID = 128
D_OUT = 64

NC = 2
NS = 16
L = 16
NW = NC * NS
EPT = E // NW
B = 80
NB = EPT // B
RPS = 624
TAIL0 = NS * RPS
TAILN = N - TAIL0


def _make_spmm(D):
    mesh = plsc.VectorSubcoreMesh(core_axis_name="c", subcore_axis_name="s")
    cp = pltpu.CompilerParams()
    if "needs_layout_passes" in pltpu.CompilerParams.__dataclass_fields__:
        cp = dataclasses.replace(cp, needs_layout_passes=False)
    if D % 128 != 0 and "use_tc_tiling_on_sc" in pltpu.CompilerParams.__dataclass_fields__:
        cp = dataclasses.replace(cp, use_tc_tiling_on_sc=False)

    @functools.partial(
        pl.kernel,
        out_type=jax.ShapeDtypeStruct((NC, N, D), jnp.float32),
        mesh=mesh,
        compiler_params=cp,
        scratch_types=[
            pltpu.VMEM((EPT,), jnp.int32),
            pltpu.VMEM((EPT,), jnp.int32),
            pltpu.VMEM((EPT,), jnp.float32),
            pltpu.VMEM((B,), jnp.int32),
            pltpu.VMEM((B,), jnp.int32),
            pltpu.VMEM((B, D), jnp.float32),
            pltpu.VMEM((B, D), jnp.float32),
            pltpu.VMEM_SHARED((N, D), jnp.float32),
            pltpu.SemaphoreType.DMA,
            pltpu.SemaphoreType.DMA,
        ],
    )
    def spmm(h_hbm, src_hbm, dst_hbm, ev_hbm, z_hbm, out_hbm,
             src_v, dst_v, ev_v, dst_a, dst_b, rows_a, rows_b,
             acc, sem_a, sem_b):
        cid = lax.axis_index("c")
        sid = lax.axis_index("s")
        wid = cid * NS + sid

        pltpu.sync_copy(src_hbm.at[pl.ds(wid * EPT, EPT)], src_v)
        pltpu.sync_copy(dst_hbm.at[pl.ds(wid * EPT, EPT)], dst_v)
        pltpu.sync_copy(ev_hbm.at[pl.ds(wid * EPT, EPT)], ev_v)

        pltpu.sync_copy(z_hbm, acc.at[pl.ds(sid * RPS, RPS)])

        @pl.when(sid == NS - 1)
        def _():
            pltpu.sync_copy(z_hbm.at[pl.ds(0, TAILN)],
                            acc.at[pl.ds(TAIL0, TAILN)])

        plsc.subcore_barrier()

        def fire(r, buf, sem):
            pltpu.async_copy(h_hbm.at[src_v.at[pl.ds(r * B, B)]], buf, sem)

        def wait(buf, sem):
            pltpu.make_async_copy(h_hbm.at[src_v.at[pl.ds(0, B)]],
                                  buf, sem).wait()

        def scale(r, buf):
            @pl.loop(0, B)
            def _(i):
                evb = plsc.load_gather(ev_v, [lax.broadcast(r * B + i, (L,))])
                for j in range(D // L):
                    sl = pl.ds(j * L, L)
                    buf[i, sl] = buf[i, sl] * evb

        def scatter(r, dst_blk, buf):
            @pl.loop(0, B, step=L)
            def _(i):
                dst_blk[pl.ds(i, L)] = dst_v[pl.ds(r * B + i, L)]

            pltpu.sync_copy(buf, acc.at[dst_blk], add=True)

        fire(0, rows_a, sem_a)

        @pl.loop(0, NB - 1, step=2)
        def _(r):
            fire(r + 1, rows_b, sem_b)
            wait(rows_a, sem_a)
            scale(r, rows_a)
            scatter(r, dst_a, rows_a)
            fire(r + 2, rows_a, sem_a)
            wait(rows_b, sem_b)
            scale(r + 1, rows_b)
            scatter(r + 1, dst_b, rows_b)

        wait(rows_a, sem_a)
        scale(NB - 1, rows_a)
        scatter(NB - 1, dst_a, rows_a)

        plsc.subcore_barrier()
        pltpu.sync_copy(acc.at[pl.ds(sid * RPS, RPS)],
                        out_hbm.at[cid].at[pl.ds(sid * RPS, RPS)])

        @pl.when(sid == NS - 1)
        def _():
            pltpu.sync_copy(acc.at[pl.ds(TAIL0, TAILN)],
                            out_hbm.at[cid].at[pl.ds(TAIL0, TAILN)])

    return spmm


_spmm_128 = _make_spmm(D_HID)
_spmm_64 = _make_spmm(D_OUT)


def _mm1_body(x_ref, w_ref, o_ref):
    o_ref[...] = jnp.dot(x_ref[...], w_ref[...],
                         preferred_element_type=jnp.float32)


def _mid_body(p_ref, b1_ref, w2_ref, o_ref):
    h1 = jnp.maximum(p_ref[0] + p_ref[1] + b1_ref[...], 0.0)
    o_ref[...] = jnp.dot(h1, w2_ref[...], preferred_element_type=jnp.float32)


def _final_body(q_ref, b2_ref, o_ref):
    o_ref[...] = q_ref[0] + q_ref[1] + b2_ref[...]


def kernel(x, edge_index, edge_vals, w1, b1, w2, b2):
    src = edge_index[0]
    dst = edge_index[1]
    ev2 = edge_vals
    z128 = jnp.zeros((RPS, D_HID), jnp.float32)
    z64 = jnp.zeros((RPS, D_OUT), jnp.float32)

    h = pl.pallas_call(
        _mm1_body,
        out_shape=jax.ShapeDtypeStruct((N, D_HID), jnp.float32),
    )(x, w1)

    p = _spmm_128(h, src, dst, ev2, z128)

    h2 = pl.pallas_call(
        _mid_body,
        out_shape=jax.ShapeDtypeStruct((N, D_OUT), jnp.float32),
    )(p, b1, w2)

    q = _spmm_64(h2, src, dst, ev2, z64)

    out = pl.pallas_call(
        _final_body,
        out_shape=jax.ShapeDtypeStruct((N, D_OUT), jnp.float32),
    )(q, b2)
    return out


# speedup vs baseline: 11.4218x; 1.2078x over previous
import dataclasses
import functools

import jax
import jax.numpy as jnp
from jax import lax
from jax.experimental import pallas as pl
from jax.experimental.pallas import tpu as pltpu
from jax.experimental.pallas import tpu_sc as plsc

N = 10000
E = 320000
D_IN = 128
D_HID = 128
D_OUT = 64

NC = 2
NS = 16
L = 16
NW = NC * NS
EPT = E // NW
B = 80
NB = EPT // B
RPS = 624
TAIL0 = NS * RPS
TAILN = N - TAIL0


def _make_spmm(D):
    mesh = plsc.VectorSubcoreMesh(core_axis_name="c", subcore_axis_name="s")
    cp = pltpu.CompilerParams()
    if "needs_layout_passes" in pltpu.CompilerParams.__dataclass_fields__:
        cp = dataclasses.replace(cp, needs_layout_passes=False)
    if D % 128 != 0 and "use_tc_tiling_on_sc" in pltpu.CompilerParams.__dataclass_fields__:
        cp = dataclasses.replace(cp, use_tc_tiling_on_sc=False)

    @functools.partial(
        pl.kernel,
        out_type=jax.ShapeDtypeStruct((NC, N, D), jnp.float32),
        mesh=mesh,
        compiler_params=cp,
        scratch_types=[
            pltpu.VMEM((EPT,), jnp.int32),
            pltpu.VMEM((EPT,), jnp.int32),
            pltpu.VMEM((EPT,), jnp.float32),
            pltpu.VMEM((B,), jnp.int32),
            pltpu.VMEM((B,), jnp.int32),
            pltpu.VMEM((B, D), jnp.float32),
            pltpu.VMEM((B, D), jnp.float32),
            pltpu.VMEM_SHARED((N, D), jnp.float32),
            pltpu.SemaphoreType.DMA,
            pltpu.SemaphoreType.DMA,
        ],
    )
    def spmm(h_hbm, src_hbm, dst_hbm, ev_hbm, z_hbm, out_hbm,
             src_v, dst_v, ev_v, dst_a, dst_b, rows_a, rows_b,
             acc, sem_a, sem_b):
        cid = lax.axis_index("c")
        sid = lax.axis_index("s")
        wid = cid * NS + sid

        pltpu.sync_copy(src_hbm.at[pl.ds(wid * EPT, EPT)], src_v)
        pltpu.sync_copy(dst_hbm.at[pl.ds(wid * EPT, EPT)], dst_v)
        pltpu.sync_copy(ev_hbm.at[pl.ds(wid * EPT, EPT)], ev_v)

        pltpu.sync_copy(z_hbm, acc.at[pl.ds(sid * RPS, RPS)])

        @pl.when(sid == NS - 1)
        def _():
            pltpu.sync_copy(z_hbm.at[pl.ds(0, TAILN)],
                            acc.at[pl.ds(TAIL0, TAILN)])

        plsc.subcore_barrier()

        def fire(r, buf, sem):
            pltpu.async_copy(h_hbm.at[src_v.at[pl.ds(r * B, B)]], buf, sem)

        def wait(buf, sem):
            pltpu.make_async_copy(h_hbm.at[src_v.at[pl.ds(0, B)]],
                                  buf, sem).wait()

        def scale(r, buf):
            @plsc.parallel_loop(0, B, unroll=4)
            def _(i):
                evb = plsc.load_gather(ev_v, [lax.broadcast(r * B + i, (L,))])
                for j in range(D // L):
                    sl = pl.ds(j * L, L)
                    buf[i, sl] = buf[i, sl] * evb

        def scatter(r, dst_blk, buf):
            @plsc.parallel_loop(0, B, step=L, unroll=5)
            def _(i):
                dst_blk[pl.ds(i, L)] = dst_v[pl.ds(r * B + i, L)]

            pltpu.sync_copy(buf, acc.at[dst_blk], add=True)

        fire(0, rows_a, sem_a)

        @pl.loop(0, NB - 1, step=2)
        def _(r):
            fire(r + 1, rows_b, sem_b)
            wait(rows_a, sem_a)
            scale(r, rows_a)
            scatter(r, dst_a, rows_a)
            fire(r + 2, rows_a, sem_a)
            wait(rows_b, sem_b)
            scale(r + 1, rows_b)
            scatter(r + 1, dst_b, rows_b)

        wait(rows_a, sem_a)
        scale(NB - 1, rows_a)
        scatter(NB - 1, dst_a, rows_a)

        plsc.subcore_barrier()
        pltpu.sync_copy(acc.at[pl.ds(sid * RPS, RPS)],
                        out_hbm.at[cid].at[pl.ds(sid * RPS, RPS)])

        @pl.when(sid == NS - 1)
        def _():
            pltpu.sync_copy(acc.at[pl.ds(TAIL0, TAILN)],
                            out_hbm.at[cid].at[pl.ds(TAIL0, TAILN)])

    return spmm


_spmm_128 = _make_spmm(D_HID)
_spmm_64 = _make_spmm(D_OUT)


def _mm1_body(x_ref, w_ref, o_ref):
    o_ref[...] = jnp.dot(x_ref[...], w_ref[...],
                         preferred_element_type=jnp.float32)


def _mid_body(p_ref, b1_ref, w2_ref, o_ref):
    h1 = jnp.maximum(p_ref[0] + p_ref[1] + b1_ref[...], 0.0)
    o_ref[...] = jnp.dot(h1, w2_ref[...], preferred_element_type=jnp.float32)


def _final_body(q_ref, b2_ref, o_ref):
    o_ref[...] = q_ref[0] + q_ref[1] + b2_ref[...]


def kernel(x, edge_index, edge_vals, w1, b1, w2, b2):
    src = edge_index[0]
    dst = edge_index[1]
    ev2 = edge_vals
    z128 = jnp.zeros((RPS, D_HID), jnp.float32)
    z64 = jnp.zeros((RPS, D_OUT), jnp.float32)

    h = pl.pallas_call(
        _mm1_body,
        out_shape=jax.ShapeDtypeStruct((N, D_HID), jnp.float32),
    )(x, w1)

    p = _spmm_128(h, src, dst, ev2, z128)

    h2 = pl.pallas_call(
        _mid_body,
        out_shape=jax.ShapeDtypeStruct((N, D_OUT), jnp.float32),
    )(p, b1, w2)

    q = _spmm_64(h2, src, dst, ev2, z64)

    out = pl.pallas_call(
        _final_body,
        out_shape=jax.ShapeDtypeStruct((N, D_OUT), jnp.float32),
    )(q, b2)
    return out
